# baseline (speedup 1.0000x reference)
import functools

import jax
import jax.numpy as jnp
from jax.experimental import pallas as pl
from jax.experimental.pallas import tpu as pltpu

_C = 1000
_B = 16384
_S = 0.1
_COEF_B = _S / (_C - 1)
_COEF_A = 1.0 - _S - _COEF_B

_BLOCK_ROWS = 512
_GRID = _B // _BLOCK_ROWS


def _loss_body(x_ref, t_ref, out_ref):
    x = x_ref[...]
    t = t_ref[...]
    m = jnp.max(x, axis=1, keepdims=True)
    e = jnp.exp(x - m)
    cols = jax.lax.broadcasted_iota(jnp.int32, x.shape, 1)
    xm = jnp.where(cols == t, x, 0.0)
    ones = jnp.ones((x.shape[1], 1), dtype=jnp.float32)
    s = jax.lax.dot(e, ones, preferred_element_type=jnp.float32)
    rowsum = jax.lax.dot(x, ones, preferred_element_type=jnp.float32)
    tgt = jax.lax.dot(xm, ones, preferred_element_type=jnp.float32)
    lse = m + jnp.log(s)
    part = lse - _COEF_A * tgt - _COEF_B * rowsum
    out_ref[0] = jnp.sum(part, axis=0, keepdims=True)


@functools.partial(jax.jit, static_argnames=("interpret",))
def _loss(logits, target, interpret=False):
    t2d = target.reshape(_B, 1)
    partials = pl.pallas_call(
        _loss_body,
        grid=(_GRID,),
        in_specs=[
            pl.BlockSpec((_BLOCK_ROWS, _C), lambda i: (i, 0)),
            pl.BlockSpec((_BLOCK_ROWS, 1), lambda i: (i, 0)),
        ],
        out_specs=pl.BlockSpec((1, 1, 1), lambda i: (i, 0, 0)),
        out_shape=jax.ShapeDtypeStruct((_GRID, 1, 1), jnp.float32),
        compiler_params=pltpu.CompilerParams(
            dimension_semantics=("parallel",),
        ),
        interpret=interpret,
    )(logits, t2d)
    return jnp.sum(partials) * (1.0 / _B)


def kernel(logits, target):
    return _loss(logits, target)
